# baseline (speedup 1.0000x reference)
import jax
import jax.numpy as jnp
from jax import lax
from jax.experimental import pallas as pl
from jax.experimental.pallas import tpu as pltpu
from jax.experimental.pallas import tpu_sc as plsc

_VOCAB = 32000
_NC, _NS, _L = 2, 16, 16


def _zero_body(out_ref):
    out_ref[...] = jnp.zeros(out_ref.shape, jnp.float32)


def _scatter_body(ids_hbm, anchor_hbm, out_ref, ids_v, anc_v, vbuf):
    b = lax.axis_index("c") * _NS + lax.axis_index("s")
    s = ids_hbm.shape[1]
    pltpu.sync_copy(ids_hbm.at[b, pl.ds(s - _L, _L)], ids_v)
    pltpu.sync_copy(anchor_hbm, anc_v.at[pl.ds(0, 1)])
    tok = (ids_v[...][_L - 1] + 1) % (_VOCAB - 1) + 1
    val = 10.0 + anc_v[...][0]
    off = (tok // _L) * _L
    vbuf[...] = jnp.where(lax.iota(jnp.int32, _L) == tok - off, val, 0.0)
    pltpu.sync_copy(vbuf, out_ref.at[b, s - 1, pl.ds(off, _L)])


def kernel(input_ids, anchor):
    batch, seq_len = input_ids.shape
    zeros = pl.pallas_call(
        _zero_body,
        grid=(batch,),
        out_specs=pl.BlockSpec((1, seq_len, _VOCAB), lambda b: (b, 0, 0)),
        out_shape=jax.ShapeDtypeStruct((batch, seq_len, _VOCAB), jnp.float32),
    )()
    mesh = plsc.VectorSubcoreMesh(
        core_axis_name="c", subcore_axis_name="s",
        num_cores=_NC, num_subcores=_NS,
    )
    scatter = pl.kernel(
        _scatter_body,
        out_type=(),
        mesh=mesh,
        scratch_types=[
            pltpu.VMEM((_L,), jnp.int32),
            pltpu.VMEM((_L,), jnp.float32),
            pltpu.VMEM((_L,), jnp.float32),
        ],
    )
    logits_ref = jax.new_ref(zeros)
    scatter(input_ids, anchor, logits_ref)
    return logits_ref[...]


# speedup vs baseline: 1.3687x; 1.3687x over previous
import jax
import jax.numpy as jnp
from jax.experimental import pallas as pl
from jax.experimental.pallas import tpu as pltpu

_VOCAB = 32000


def _body(ids_ref, anchor_ref, out_ref):
    b = pl.program_id(0)
    s = out_ref.shape[1]
    tok = (ids_ref[b, s - 1] + 1) % (_VOCAB - 1) + 1
    val = 10.0 + anchor_ref[0]
    out_ref[...] = jnp.zeros(out_ref.shape, jnp.float32)
    col = jax.lax.broadcasted_iota(jnp.int32, (1, _VOCAB), 1)
    out_ref[:, s - 1, :] = jnp.where(col == tok, val, 0.0)


def kernel(input_ids, anchor):
    batch, seq_len = input_ids.shape
    grid_spec = pltpu.PrefetchScalarGridSpec(
        num_scalar_prefetch=2,
        grid=(batch,),
        in_specs=[],
        out_specs=pl.BlockSpec(
            (1, seq_len, _VOCAB), lambda b, ids, anc: (b, 0, 0)
        ),
    )
    return pl.pallas_call(
        _body,
        grid_spec=grid_spec,
        out_shape=jax.ShapeDtypeStruct((batch, seq_len, _VOCAB), jnp.float32),
    )(input_ids, anchor)
